# baseline (speedup 1.0000x reference)
import jax
import jax.numpy as jnp
import numpy as np

TOP_K = 48
NUM_RBF = 16
MAX_REL = 32
NUM_POS = 16
EDGE_FEATURES = 128


def _gather_edges(edges, neighbor_idx):
    neighbors = jnp.tile(jnp.expand_dims(neighbor_idx, -1), [1, 1, 1, edges.shape[-1]])
    return jnp.take_along_axis(edges, neighbors, 2)


def _rbf(D):
    D_mu = jnp.linspace(2.0, 22.0, NUM_RBF).reshape([1, 1, 1, -1])
    D_sigma = (22.0 - 2.0) / NUM_RBF
    return jnp.exp(-((D[..., None] - D_mu) / D_sigma) ** 2)


def _get_rbf(A, B, E_idx):
    D = jnp.sqrt(jnp.sum((A[:, :, None, :] - B[:, None, :, :]) ** 2, -1) + 1e-06)
    Dn = _gather_edges(D[:, :, :, None], E_idx)[:, :, :, 0]
    return _rbf(Dn)


def _manual_approx_min_k(D_adjust, k):
    neg_top, E_idx = jax.lax.top_k(-D_adjust, k)
    return -neg_top, E_idx.astype(jnp.int32)


def kernel(X, mask, residue_idx, chain_labels, W_pos, b_pos, W_edge, ln_scale, ln_offset):
    b = X[:, :, 1, :] - X[:, :, 0, :]
    c = X[:, :, 2, :] - X[:, :, 1, :]
    a = jnp.cross(b, c)
    Cb = -0.58273431 * a + 0.56802827 * b - 0.54067466 * c + X[:, :, 1, :]
    Ca = X[:, :, 1, :]
    N = X[:, :, 0, :]
    C = X[:, :, 2, :]
    O = X[:, :, 3, :]
    mask_2D = jnp.expand_dims(mask, 1) * jnp.expand_dims(mask, 2)
    dX = jnp.expand_dims(Ca, 1) - jnp.expand_dims(Ca, 2)
    D = mask_2D * jnp.sqrt(jnp.sum(dX ** 2, 3) + 1e-06)
    D_max = jnp.max(D, -1, keepdims=True)
    D_adjust = D + (1.0 - mask_2D) * D_max
    k = int(np.minimum(TOP_K, Ca.shape[1]))
    D_neighbors, E_idx = _manual_approx_min_k(D_adjust, k)
    RBF_all = [_rbf(D_neighbors)]
    pairs = [(N, N), (C, C), (O, O), (Cb, Cb), (Ca, N), (Ca, C), (Ca, O), (Ca, Cb), (N, C), (N, O), (N, Cb), (Cb, C), (Cb, O), (O, C), (N, Ca), (C, Ca), (O, Ca), (Cb, Ca), (C, N), (O, N), (Cb, N), (C, Cb), (O, Cb), (C, O)]
    for A_, B_ in pairs:
        RBF_all.append(_get_rbf(A_, B_, E_idx))
    RBF_all = jnp.concatenate(tuple(RBF_all), axis=-1)
    offset = residue_idx[:, :, None] - residue_idx[:, None, :]
    offset = _gather_edges(offset[:, :, :, None], E_idx)[:, :, :, 0]
    d_chains = (chain_labels[:, :, None] - chain_labels[:, None, :] == 0).astype(jnp.int32)
    E_chains = _gather_edges(d_chains[:, :, :, None], E_idx)[:, :, :, 0]
    d = jnp.clip(offset + MAX_REL, 0, 2 * MAX_REL) * E_chains + (1 - E_chains) * (2 * MAX_REL + 1)
    d_onehot = jax.nn.one_hot(d, 2 * MAX_REL + 2)
    E_pos = d_onehot.astype(jnp.float32) @ W_pos + b_pos
    E = jnp.concatenate((E_pos, RBF_all), -1)
    E = E @ W_edge
    mu = jnp.mean(E, -1, keepdims=True)
    var = jnp.var(E, -1, keepdims=True)
    E = (E - mu) / jnp.sqrt(var + 1e-05) * ln_scale + ln_offset
    return E, E_idx
